# baseline (speedup 1.0000x reference)
import functools

import jax
import jax.numpy as jnp
from jax import lax
from jax.experimental import pallas as pl
from jax.experimental.pallas import tpu as pltpu
from jax.experimental.pallas import tpu_sc as plsc

_CH = 80
_NBUF = 3


def _leaky(x, slope):
    return jnp.where(x >= 0, x, slope * x)


def _tc1_body(nf_ref, w1a_ref, w1b_ref, sp_ref, dp_ref):
    nf = nf_ref[...]
    sp_ref[...] = jnp.dot(nf, w1a_ref[...], preferred_element_type=jnp.float32)
    dp_ref[...] = jnp.dot(nf, w1b_ref[...], preferred_element_type=jnp.float32)


def _make_tc1(N, D, blk):
    grid = N // blk
    return pl.pallas_call(
        _tc1_body,
        grid=(grid,),
        in_specs=[
            pl.BlockSpec((blk, D), lambda i: (i, 0)),
            pl.BlockSpec((D, D), lambda i: (0, 0)),
            pl.BlockSpec((D, D), lambda i: (0, 0)),
        ],
        out_specs=[
            pl.BlockSpec((blk, D), lambda i: (i, 0)),
            pl.BlockSpec((blk, D), lambda i: (i, 0)),
        ],
        out_shape=[
            jax.ShapeDtypeStruct((N, D), jnp.float32),
            jax.ShapeDtypeStruct((N, D), jnp.float32),
        ],
    )


def _make_sc1(E, D, NC, NS):
    NW = NC * NS
    EPW = E // NW
    CH, NB = _CH, _NBUF
    NCHUNK = EPW // CH
    NSUP = NCHUNK // NB
    NTAIL = NCHUNK - NSUP * NB
    assert EPW % CH == 0

    mesh = plsc.VectorSubcoreMesh(core_axis_name="c", subcore_axis_name="s")

    @functools.partial(
        pl.kernel,
        mesh=mesh,
        out_type=jax.ShapeDtypeStruct((E, D), jnp.float32),
        compiler_params=pltpu.CompilerParams(use_tc_tiling_on_sc=False,
                                             needs_layout_passes=False),
        scratch_types=[
            pltpu.VMEM((NCHUNK, CH), jnp.int32),
            pltpu.VMEM((NCHUNK, CH), jnp.int32),
            pltpu.VMEM((NB * CH,), jnp.int32),
            pltpu.VMEM((NB * CH, D), jnp.float32),
            pltpu.VMEM((NB * CH, D), jnp.float32),
            pltpu.VMEM_SHARED((NS * NB * CH, D), jnp.float32),
            pltpu.SemaphoreType.DMA((NB,)),
            pltpu.SemaphoreType.DMA((NB,)),
        ],
    )
    def sc1(sp_hbm, dp_hbm, src_hbm, dst_hbm, out_hbm,
            idxs, idxd, identb, bufA, bufB, acc,
            gsA, gsB):
        c = lax.axis_index("c")
        s = lax.axis_index("s")
        wid = s * NC + c
        base = wid * EPW
        sbase = s * NB * CH
        pltpu.sync_copy(src_hbm.at[wid], idxs)
        pltpu.sync_copy(dst_hbm.at[wid], idxd)
        for k in range((NB * CH) // 16):
            identb[pl.ds(k * 16, 16)] = (
                lax.iota(jnp.int32, 16) + (sbase + 16 * k))

        for j in range(NB):
            pltpu.async_copy(sp_hbm.at[idxs.at[j]],
                             bufA.at[pl.ds(j * CH, CH)], gsA.at[j])
            pltpu.async_copy(dp_hbm.at[idxd.at[j]],
                             bufB.at[pl.ds(j * CH, CH)], gsB.at[j])

        def super_body(sidx, carry):
            for j in range(NB):
                pltpu.make_async_copy(sp_hbm.at[idxs.at[0]],
                                      bufA.at[pl.ds(j * CH, CH)],
                                      gsA.at[j]).wait()
                pltpu.make_async_copy(dp_hbm.at[idxd.at[0]],
                                      bufB.at[pl.ds(j * CH, CH)],
                                      gsB.at[j]).wait()
            off = base + sidx * (NB * CH)
            pltpu.sync_copy(bufA, acc.at[pl.ds(sbase, NB * CH)])
            pltpu.sync_copy(bufB, acc.at[identb], add=True)
            pltpu.sync_copy(acc.at[pl.ds(sbase, NB * CH)],
                            out_hbm.at[pl.ds(off, NB * CH)])
            for j in range(NB):
                ch = sidx * NB + j

                @pl.when(sidx < NSUP - 1)
                def _():
                    nch = ch + NB
                    pltpu.async_copy(sp_hbm.at[idxs.at[nch]],
                                     bufA.at[pl.ds(j * CH, CH)], gsA.at[j])
                    pltpu.async_copy(dp_hbm.at[idxd.at[nch]],
                                     bufB.at[pl.ds(j * CH, CH)], gsB.at[j])
            return carry

        lax.fori_loop(0, NSUP, super_body, 0)
        for t in range(NTAIL):
            ch = NSUP * NB + t
            off = base + ch * CH
            cpA = pltpu.async_copy(sp_hbm.at[idxs.at[ch]],
                                   bufA.at[pl.ds(0, CH)], gsA.at[0])
            cpB = pltpu.async_copy(dp_hbm.at[idxd.at[ch]],
                                   bufB.at[pl.ds(0, CH)], gsB.at[0])
            cpA.wait()
            cpB.wait()
            pltpu.sync_copy(bufA.at[pl.ds(0, CH)], acc.at[pl.ds(sbase, CH)])
            pltpu.sync_copy(bufB.at[pl.ds(0, CH)],
                            acc.at[identb.at[pl.ds(0, CH)]], add=True)
            pltpu.sync_copy(acc.at[pl.ds(sbase, CH)], out_hbm.at[pl.ds(off, CH)])

    return sc1


def _make_tc2_body(D, W):
    def body(g_ref, ef_ref, w1c_ref, w2_ref, w3_ref, e_ref, p_ref):
        x = g_ref[...] + jnp.dot(ef_ref[...], w1c_ref[...],
                                 preferred_element_type=jnp.float32)
        e = _leaky(x, 1e-4)
        e_ref[...] = e
        w3e = _leaky(jnp.dot(e, w3_ref[...], preferred_element_type=jnp.float32),
                     0.01)
        lg = _leaky(jnp.dot(e, w2_ref[...], preferred_element_type=jnp.float32),
                    1e-4)
        ex = jnp.exp(lg)
        p_ref[:, :D] = w3e * ex
        p_ref[:, D:D + 1] = ex
        p_ref[:, D + 1:] = jnp.zeros((ex.shape[0], W - D - 1), jnp.float32)
    return body


def _make_tc2(E, D, W, blk):
    grid = E // blk
    return pl.pallas_call(
        _make_tc2_body(D, W),
        grid=(grid,),
        in_specs=[
            pl.BlockSpec((blk, D), lambda i: (i, 0)),
            pl.BlockSpec((blk, D), lambda i: (i, 0)),
            pl.BlockSpec((D, D), lambda i: (0, 0)),
            pl.BlockSpec((D, 1), lambda i: (0, 0)),
            pl.BlockSpec((D, D), lambda i: (0, 0)),
        ],
        out_specs=[
            pl.BlockSpec((blk, D), lambda i: (i, 0)),
            pl.BlockSpec((blk, W), lambda i: (i, 0)),
        ],
        out_shape=[
            jax.ShapeDtypeStruct((E, D), jnp.float32),
            jax.ShapeDtypeStruct((E, W), jnp.float32),
        ],
    )


def _make_sc2(E, N, D, W, NC, NS):
    NW = NC * NS
    EPW = E // NW
    CH, NB = _CH, 2
    NCHUNK = EPW // CH
    NSUP = NCHUNK // NB
    NTAIL = NCHUNK - NSUP * NB
    NCH_N = N // CH
    assert N % CH == 0
    mesh = plsc.VectorSubcoreMesh(core_axis_name="c", subcore_axis_name="s")

    @functools.partial(
        pl.kernel,
        mesh=mesh,
        out_type=jax.ShapeDtypeStruct((NC, N, W), jnp.float32),
        compiler_params=pltpu.CompilerParams(use_tc_tiling_on_sc=False,
                                             needs_layout_passes=False),
        scratch_types=[
            pltpu.VMEM((EPW,), jnp.int32),
            pltpu.VMEM((NB * CH, W), jnp.float32),
            pltpu.VMEM_SHARED((N, W), jnp.float32),
            pltpu.SemaphoreType.DMA((NB,)),
        ],
    )
    def sc2(p_hbm, dst_hbm, out_hbm, idxd, bufP, mtab, lsem):
        c = lax.axis_index("c")
        s = lax.axis_index("s")
        wid = s * NC + c
        base = wid * EPW
        nstripe = (NCH_N - s + NS - 1) // NS

        pltpu.sync_copy(dst_hbm.at[wid], idxd)

        zf = jnp.zeros((16,), jnp.float32)

        def zrow(r, carry):
            for k in range(W // 16):
                bufP[r, pl.ds(k * 16, 16)] = zf
            return carry

        lax.fori_loop(0, CH, zrow, 0)

        def zstep(j, carry):
            pltpu.sync_copy(bufP.at[pl.ds(0, CH)],
                            mtab.at[pl.ds((s + j * NS) * CH, CH)])
            return carry

        lax.fori_loop(0, nstripe, zstep, 0)
        plsc.subcore_barrier()

        for j in range(NB):
            pltpu.async_copy(p_hbm.at[pl.ds(base + j * CH, CH)],
                             bufP.at[pl.ds(j * CH, CH)], lsem.at[j])

        def super_body(sidx, carry):
            for j in range(NB):
                pltpu.make_async_copy(p_hbm.at[pl.ds(base, CH)],
                                      bufP.at[pl.ds(j * CH, CH)],
                                      lsem.at[j]).wait()
            pltpu.sync_copy(
                bufP, mtab.at[idxd.at[pl.ds(sidx * (NB * CH), NB * CH)]],
                add=True)
            for j in range(NB):
                ch = sidx * NB + j

                @pl.when(sidx < NSUP - 1)
                def _():
                    off = base + (ch + NB) * CH
                    pltpu.async_copy(p_hbm.at[pl.ds(off, CH)],
                                     bufP.at[pl.ds(j * CH, CH)], lsem.at[j])
            return carry

        lax.fori_loop(0, NSUP, super_body, 0)
        for t in range(NTAIL):
            ch = NSUP * NB + t
            off = base + ch * CH
            pltpu.sync_copy(p_hbm.at[pl.ds(off, CH)], bufP.at[pl.ds(0, CH)])
            pltpu.sync_copy(bufP.at[pl.ds(0, CH)],
                            mtab.at[idxd.at[pl.ds(ch * CH, CH)]], add=True)
        plsc.subcore_barrier()

        def wstep(j, carry):
            r0 = (s + j * NS) * CH
            pltpu.sync_copy(mtab.at[pl.ds(r0, CH)],
                            out_hbm.at[c, pl.ds(r0, CH)])
            return carry

        lax.fori_loop(0, nstripe, wstep, 0)

    return sc2


def _make_tc3_body(D):
    def body(m0_ref, m1_ref, nf_ref, wih_ref, whh_ref, bih_ref, bhh_ref, h_ref):
        msum = m0_ref[:, :D] + m1_ref[:, :D]
        den = m0_ref[:, D:D + 1] + m1_ref[:, D:D + 1]
        mdiv = jnp.where(den > 0, msum / jnp.where(den > 0, den, 1.0), 0.0)
        m = jnp.where(mdiv > 0, mdiv, jnp.exp(mdiv) - 1.0)
        nf = nf_ref[...]
        gi = jnp.dot(m, wih_ref[...], preferred_element_type=jnp.float32) + bih_ref[...]
        gh = jnp.dot(nf, whh_ref[...], preferred_element_type=jnp.float32) + bhh_ref[...]
        r = jax.nn.sigmoid(gi[:, :D] + gh[:, :D])
        z = jax.nn.sigmoid(gi[:, D:2 * D] + gh[:, D:2 * D])
        n = jnp.tanh(gi[:, 2 * D:] + r * gh[:, 2 * D:])
        h_ref[...] = jnp.maximum((1.0 - z) * n + z * nf, 0.0)
    return body


def _make_tc3(N, D, W, blk):
    grid = N // blk
    return pl.pallas_call(
        _make_tc3_body(D),
        grid=(grid,),
        in_specs=[
            pl.BlockSpec((blk, W), lambda i: (i, 0)),
            pl.BlockSpec((blk, W), lambda i: (i, 0)),
            pl.BlockSpec((blk, D), lambda i: (i, 0)),
            pl.BlockSpec((D, 3 * D), lambda i: (0, 0)),
            pl.BlockSpec((D, 3 * D), lambda i: (0, 0)),
            pl.BlockSpec((1, 3 * D), lambda i: (0, 0)),
            pl.BlockSpec((1, 3 * D), lambda i: (0, 0)),
        ],
        out_specs=pl.BlockSpec((blk, D), lambda i: (i, 0)),
        out_shape=jax.ShapeDtypeStruct((N, D), jnp.float32),
    )


def kernel(nfeats, efeats, edge_index, W1, W2, W3, W_ih, W_hh, b_ih, b_hh):
    N, D = nfeats.shape
    E = efeats.shape[0]
    W = D + 16
    info = plsc.get_sparse_core_info()
    NC, NS = info.num_cores, info.num_subcores
    NW = NC * NS
    NCHUNK = (E // NW) // _CH

    src = edge_index[0].reshape(NW, NCHUNK, _CH)
    dst = edge_index[1].reshape(NW, NCHUNK, _CH)

    sp, dp = _make_tc1(N, D, 2000)(nfeats, W1[:D], W1[D:2 * D])
    g = _make_sc1(E, D, NC, NS)(sp, dp, src, dst)
    e, p = _make_tc2(E, D, W, 2000)(g, efeats, W1[2 * D:], W2, W3)
    mtab = _make_sc2(E, N, D, W, NC, NS)(p, edge_index[1].reshape(NW, E // NW))
    h_new = _make_tc3(N, D, W, 2000)(
        mtab[0], mtab[1], nfeats, W_ih, W_hh,
        b_ih.reshape(1, 3 * D), b_hh.reshape(1, 3 * D))
    return (h_new, e)


# speedup vs baseline: 1.0691x; 1.0691x over previous
import functools

import jax
import jax.numpy as jnp
from jax import lax
from jax.experimental import pallas as pl
from jax.experimental.pallas import tpu as pltpu
from jax.experimental.pallas import tpu_sc as plsc

_CH = 80
_NBUF = 3


def _leaky(x, slope):
    return jnp.where(x >= 0, x, slope * x)


def _tc1_body(nf_ref, w1a_ref, w1b_ref, sp_ref, dp_ref):
    nf = nf_ref[...]
    sp_ref[...] = jnp.dot(nf, w1a_ref[...], preferred_element_type=jnp.float32)
    dp_ref[...] = jnp.dot(nf, w1b_ref[...], preferred_element_type=jnp.float32)


def _make_tc1(N, D, blk):
    grid = N // blk
    return pl.pallas_call(
        _tc1_body,
        grid=(grid,),
        in_specs=[
            pl.BlockSpec((blk, D), lambda i: (i, 0)),
            pl.BlockSpec((D, D), lambda i: (0, 0)),
            pl.BlockSpec((D, D), lambda i: (0, 0)),
        ],
        out_specs=[
            pl.BlockSpec((blk, D), lambda i: (i, 0)),
            pl.BlockSpec((blk, D), lambda i: (i, 0)),
        ],
        out_shape=[
            jax.ShapeDtypeStruct((N, D), jnp.float32),
            jax.ShapeDtypeStruct((N, D), jnp.float32),
        ],
    )


def _make_sc1(E, D, NC, NS):
    NW = NC * NS
    EPW = E // NW
    CH, NB = _CH, _NBUF
    NCHUNK = EPW // CH
    NSUP = NCHUNK // NB
    NTAIL = NCHUNK - NSUP * NB
    assert EPW % CH == 0

    mesh = plsc.VectorSubcoreMesh(core_axis_name="c", subcore_axis_name="s")

    @functools.partial(
        pl.kernel,
        mesh=mesh,
        out_type=jax.ShapeDtypeStruct((E, D), jnp.float32),
        compiler_params=pltpu.CompilerParams(use_tc_tiling_on_sc=False,
                                             needs_layout_passes=False),
        scratch_types=[
            pltpu.VMEM((NCHUNK, CH), jnp.int32),
            pltpu.VMEM((NCHUNK, CH), jnp.int32),
            pltpu.VMEM((NB * CH,), jnp.int32),
            pltpu.VMEM((NB * CH, D), jnp.float32),
            pltpu.VMEM((NB * CH, D), jnp.float32),
            pltpu.VMEM_SHARED((NS * NB * CH, D), jnp.float32),
            pltpu.SemaphoreType.DMA((NB,)),
            pltpu.SemaphoreType.DMA((NB,)),
        ],
    )
    def sc1(sp_hbm, dp_hbm, src_hbm, dst_hbm, out_hbm,
            idxs, idxd, identb, bufA, bufB, acc,
            gsA, gsB):
        c = lax.axis_index("c")
        s = lax.axis_index("s")
        wid = s * NC + c
        base = wid * EPW
        sbase = s * NB * CH
        pltpu.sync_copy(src_hbm.at[wid], idxs)
        pltpu.sync_copy(dst_hbm.at[wid], idxd)
        for k in range((NB * CH) // 16):
            identb[pl.ds(k * 16, 16)] = (
                lax.iota(jnp.int32, 16) + (sbase + 16 * k))

        for j in range(NB):
            pltpu.async_copy(sp_hbm.at[idxs.at[j]],
                             bufA.at[pl.ds(j * CH, CH)], gsA.at[j])
            pltpu.async_copy(dp_hbm.at[idxd.at[j]],
                             bufB.at[pl.ds(j * CH, CH)], gsB.at[j])

        def super_body(sidx, carry):
            for j in range(NB):
                pltpu.make_async_copy(sp_hbm.at[idxs.at[0]],
                                      bufA.at[pl.ds(j * CH, CH)],
                                      gsA.at[j]).wait()
                pltpu.make_async_copy(dp_hbm.at[idxd.at[0]],
                                      bufB.at[pl.ds(j * CH, CH)],
                                      gsB.at[j]).wait()
            for j in range(NB):
                ch = sidx * NB + j
                off = base + ch * CH
                pltpu.sync_copy(bufA.at[pl.ds(j * CH, CH)],
                                acc.at[pl.ds(sbase + j * CH, CH)])
                pltpu.sync_copy(bufB.at[pl.ds(j * CH, CH)],
                                acc.at[identb.at[pl.ds(j * CH, CH)]], add=True)
                pltpu.sync_copy(acc.at[pl.ds(sbase + j * CH, CH)],
                                out_hbm.at[pl.ds(off, CH)])

                @pl.when(sidx < NSUP - 1)
                def _():
                    nch = ch + NB
                    pltpu.async_copy(sp_hbm.at[idxs.at[nch]],
                                     bufA.at[pl.ds(j * CH, CH)], gsA.at[j])
                    pltpu.async_copy(dp_hbm.at[idxd.at[nch]],
                                     bufB.at[pl.ds(j * CH, CH)], gsB.at[j])
            return carry

        lax.fori_loop(0, NSUP, super_body, 0)
        for t in range(NTAIL):
            ch = NSUP * NB + t
            off = base + ch * CH
            cpA = pltpu.async_copy(sp_hbm.at[idxs.at[ch]],
                                   bufA.at[pl.ds(0, CH)], gsA.at[0])
            cpB = pltpu.async_copy(dp_hbm.at[idxd.at[ch]],
                                   bufB.at[pl.ds(0, CH)], gsB.at[0])
            cpA.wait()
            cpB.wait()
            pltpu.sync_copy(bufA.at[pl.ds(0, CH)], acc.at[pl.ds(sbase, CH)])
            pltpu.sync_copy(bufB.at[pl.ds(0, CH)],
                            acc.at[identb.at[pl.ds(0, CH)]], add=True)
            pltpu.sync_copy(acc.at[pl.ds(sbase, CH)], out_hbm.at[pl.ds(off, CH)])

    return sc1


def _make_tc2_body(D, W):
    def body(g_ref, ef_ref, w1c_ref, w2_ref, w3_ref, e_ref, p_ref):
        x = g_ref[...] + jnp.dot(ef_ref[...], w1c_ref[...],
                                 preferred_element_type=jnp.float32)
        e = _leaky(x, 1e-4)
        e_ref[...] = e
        w3e = _leaky(jnp.dot(e, w3_ref[...], preferred_element_type=jnp.float32),
                     0.01)
        lg = _leaky(jnp.dot(e, w2_ref[...], preferred_element_type=jnp.float32),
                    1e-4)
        ex = jnp.exp(lg)
        p_ref[:, :D] = w3e * ex
        p_ref[:, D:D + 1] = ex
        p_ref[:, D + 1:] = jnp.zeros((ex.shape[0], W - D - 1), jnp.float32)
    return body


def _make_tc2(E, D, W, blk):
    grid = E // blk
    return pl.pallas_call(
        _make_tc2_body(D, W),
        grid=(grid,),
        in_specs=[
            pl.BlockSpec((blk, D), lambda i: (i, 0)),
            pl.BlockSpec((blk, D), lambda i: (i, 0)),
            pl.BlockSpec((D, D), lambda i: (0, 0)),
            pl.BlockSpec((D, 1), lambda i: (0, 0)),
            pl.BlockSpec((D, D), lambda i: (0, 0)),
        ],
        out_specs=[
            pl.BlockSpec((blk, D), lambda i: (i, 0)),
            pl.BlockSpec((blk, W), lambda i: (i, 0)),
        ],
        out_shape=[
            jax.ShapeDtypeStruct((E, D), jnp.float32),
            jax.ShapeDtypeStruct((E, W), jnp.float32),
        ],
    )


def _make_sc2(E, N, D, W, NC, NS):
    NW = NC * NS
    EPW = E // NW
    CH, NB = _CH, 2
    NCHUNK = EPW // CH
    NSUP = NCHUNK // NB
    NTAIL = NCHUNK - NSUP * NB
    NCH_N = N // CH
    assert N % CH == 0
    mesh = plsc.VectorSubcoreMesh(core_axis_name="c", subcore_axis_name="s")

    @functools.partial(
        pl.kernel,
        mesh=mesh,
        out_type=jax.ShapeDtypeStruct((NC, N, W), jnp.float32),
        compiler_params=pltpu.CompilerParams(use_tc_tiling_on_sc=False,
                                             needs_layout_passes=False),
        scratch_types=[
            pltpu.VMEM((EPW,), jnp.int32),
            pltpu.VMEM((NB * CH, W), jnp.float32),
            pltpu.VMEM_SHARED((N, W), jnp.float32),
            pltpu.SemaphoreType.DMA((NB,)),
        ],
    )
    def sc2(p_hbm, dst_hbm, out_hbm, idxd, bufP, mtab, lsem):
        c = lax.axis_index("c")
        s = lax.axis_index("s")
        wid = s * NC + c
        base = wid * EPW
        nstripe = (NCH_N - s + NS - 1) // NS

        pltpu.sync_copy(dst_hbm.at[wid], idxd)

        zf = jnp.zeros((16,), jnp.float32)

        def zrow(r, carry):
            for k in range(W // 16):
                bufP[r, pl.ds(k * 16, 16)] = zf
            return carry

        lax.fori_loop(0, CH, zrow, 0)

        def zstep(j, carry):
            pltpu.sync_copy(bufP.at[pl.ds(0, CH)],
                            mtab.at[pl.ds((s + j * NS) * CH, CH)])
            return carry

        lax.fori_loop(0, nstripe, zstep, 0)
        plsc.subcore_barrier()

        for j in range(NB):
            pltpu.async_copy(p_hbm.at[pl.ds(base + j * CH, CH)],
                             bufP.at[pl.ds(j * CH, CH)], lsem.at[j])

        def super_body(sidx, carry):
            for j in range(NB):
                ch = sidx * NB + j
                pltpu.make_async_copy(p_hbm.at[pl.ds(base, CH)],
                                      bufP.at[pl.ds(j * CH, CH)],
                                      lsem.at[j]).wait()
                pltpu.sync_copy(bufP.at[pl.ds(j * CH, CH)],
                                mtab.at[idxd.at[pl.ds(ch * CH, CH)]],
                                add=True)

                @pl.when(sidx < NSUP - 1)
                def _():
                    off = base + (ch + NB) * CH
                    pltpu.async_copy(p_hbm.at[pl.ds(off, CH)],
                                     bufP.at[pl.ds(j * CH, CH)], lsem.at[j])
            return carry

        lax.fori_loop(0, NSUP, super_body, 0)
        for t in range(NTAIL):
            ch = NSUP * NB + t
            off = base + ch * CH
            pltpu.sync_copy(p_hbm.at[pl.ds(off, CH)], bufP.at[pl.ds(0, CH)])
            pltpu.sync_copy(bufP.at[pl.ds(0, CH)],
                            mtab.at[idxd.at[pl.ds(ch * CH, CH)]], add=True)
        plsc.subcore_barrier()

        def wstep(j, carry):
            r0 = (s + j * NS) * CH
            pltpu.sync_copy(mtab.at[pl.ds(r0, CH)],
                            out_hbm.at[c, pl.ds(r0, CH)])
            return carry

        lax.fori_loop(0, nstripe, wstep, 0)

    return sc2


def _make_tc3_body(D):
    def body(m0_ref, m1_ref, nf_ref, wih_ref, whh_ref, bih_ref, bhh_ref, h_ref):
        msum = m0_ref[:, :D] + m1_ref[:, :D]
        den = m0_ref[:, D:D + 1] + m1_ref[:, D:D + 1]
        mdiv = jnp.where(den > 0, msum / jnp.where(den > 0, den, 1.0), 0.0)
        m = jnp.where(mdiv > 0, mdiv, jnp.exp(mdiv) - 1.0)
        nf = nf_ref[...]
        gi = jnp.dot(m, wih_ref[...], preferred_element_type=jnp.float32) + bih_ref[...]
        gh = jnp.dot(nf, whh_ref[...], preferred_element_type=jnp.float32) + bhh_ref[...]
        r = jax.nn.sigmoid(gi[:, :D] + gh[:, :D])
        z = jax.nn.sigmoid(gi[:, D:2 * D] + gh[:, D:2 * D])
        n = jnp.tanh(gi[:, 2 * D:] + r * gh[:, 2 * D:])
        h_ref[...] = jnp.maximum((1.0 - z) * n + z * nf, 0.0)
    return body


def _make_tc3(N, D, W, blk):
    grid = N // blk
    return pl.pallas_call(
        _make_tc3_body(D),
        grid=(grid,),
        in_specs=[
            pl.BlockSpec((blk, W), lambda i: (i, 0)),
            pl.BlockSpec((blk, W), lambda i: (i, 0)),
            pl.BlockSpec((blk, D), lambda i: (i, 0)),
            pl.BlockSpec((D, 3 * D), lambda i: (0, 0)),
            pl.BlockSpec((D, 3 * D), lambda i: (0, 0)),
            pl.BlockSpec((1, 3 * D), lambda i: (0, 0)),
            pl.BlockSpec((1, 3 * D), lambda i: (0, 0)),
        ],
        out_specs=pl.BlockSpec((blk, D), lambda i: (i, 0)),
        out_shape=jax.ShapeDtypeStruct((N, D), jnp.float32),
    )


def kernel(nfeats, efeats, edge_index, W1, W2, W3, W_ih, W_hh, b_ih, b_hh):
    N, D = nfeats.shape
    E = efeats.shape[0]
    W = D + 16
    info = plsc.get_sparse_core_info()
    NC, NS = info.num_cores, info.num_subcores
    NW = NC * NS
    NCHUNK = (E // NW) // _CH

    src = edge_index[0].reshape(NW, NCHUNK, _CH)
    dst = edge_index[1].reshape(NW, NCHUNK, _CH)

    sp, dp = _make_tc1(N, D, 2000)(nfeats, W1[:D], W1[D:2 * D])
    g = _make_sc1(E, D, NC, NS)(sp, dp, src, dst)
    e, p = _make_tc2(E, D, W, 2000)(g, efeats, W1[2 * D:], W2, W3)
    mtab = _make_sc2(E, N, D, W, NC, NS)(p, edge_index[1].reshape(NW, E // NW))
    h_new = _make_tc3(N, D, W, 2000)(
        mtab[0], mtab[1], nfeats, W_ih, W_hh,
        b_ih.reshape(1, 3 * D), b_hh.reshape(1, 3 * D))
    return (h_new, e)


# speedup vs baseline: 1.0933x; 1.0227x over previous
import functools

import jax
import jax.numpy as jnp
from jax import lax
from jax.experimental import pallas as pl
from jax.experimental.pallas import tpu as pltpu
from jax.experimental.pallas import tpu_sc as plsc

_CH = 80
_NBUF = 3


def _leaky(x, slope):
    return jnp.where(x >= 0, x, slope * x)


def _tc1_body(nf_ref, w1a_ref, w1b_ref, sp_ref, dp_ref):
    nf = nf_ref[...]
    sp_ref[...] = jnp.dot(nf, w1a_ref[...], preferred_element_type=jnp.float32)
    dp_ref[...] = jnp.dot(nf, w1b_ref[...], preferred_element_type=jnp.float32)


def _make_tc1(N, D, blk):
    grid = N // blk
    return pl.pallas_call(
        _tc1_body,
        grid=(grid,),
        in_specs=[
            pl.BlockSpec((blk, D), lambda i: (i, 0)),
            pl.BlockSpec((D, D), lambda i: (0, 0)),
            pl.BlockSpec((D, D), lambda i: (0, 0)),
        ],
        out_specs=[
            pl.BlockSpec((blk, D), lambda i: (i, 0)),
            pl.BlockSpec((blk, D), lambda i: (i, 0)),
        ],
        out_shape=[
            jax.ShapeDtypeStruct((N, D), jnp.float32),
            jax.ShapeDtypeStruct((N, D), jnp.float32),
        ],
    )


def _make_sc1(E, D, NC, NS):
    NW = NC * NS
    EPW = E // NW
    CH, NB = _CH, _NBUF
    NCHUNK = EPW // CH
    NSUP = NCHUNK // NB
    NTAIL = NCHUNK - NSUP * NB
    assert EPW % CH == 0

    mesh = plsc.VectorSubcoreMesh(core_axis_name="c", subcore_axis_name="s")

    @functools.partial(
        pl.kernel,
        mesh=mesh,
        out_type=jax.ShapeDtypeStruct((E, D), jnp.float32),
        compiler_params=pltpu.CompilerParams(use_tc_tiling_on_sc=False,
                                             needs_layout_passes=False),
        scratch_types=[
            pltpu.VMEM((NCHUNK, CH), jnp.int32),
            pltpu.VMEM((NCHUNK, CH), jnp.int32),
            pltpu.VMEM((NB * CH,), jnp.int32),
            pltpu.VMEM((NB * CH, D), jnp.float32),
            pltpu.VMEM((NB * CH, D), jnp.float32),
            pltpu.VMEM_SHARED((NS * NB * CH, D), jnp.float32),
            pltpu.SemaphoreType.DMA((NB,)),
            pltpu.SemaphoreType.DMA((NB,)),
        ],
    )
    def sc1(sp_hbm, dp_hbm, src_hbm, dst_hbm, out_hbm,
            idxs, idxd, identb, bufA, bufB, acc,
            gsA, gsB):
        c = lax.axis_index("c")
        s = lax.axis_index("s")
        wid = s * NC + c
        base = wid * EPW
        sbase = s * NB * CH
        pltpu.sync_copy(src_hbm.at[wid], idxs)
        pltpu.sync_copy(dst_hbm.at[wid], idxd)
        for k in range((NB * CH) // 16):
            identb[pl.ds(k * 16, 16)] = (
                lax.iota(jnp.int32, 16) + (sbase + 16 * k))

        for j in range(NB):
            pltpu.async_copy(sp_hbm.at[idxs.at[j]],
                             bufA.at[pl.ds(j * CH, CH)], gsA.at[j])
            pltpu.async_copy(dp_hbm.at[idxd.at[j]],
                             bufB.at[pl.ds(j * CH, CH)], gsB.at[j])

        def super_body(sidx, carry):
            for j in range(NB):
                pltpu.make_async_copy(sp_hbm.at[idxs.at[0]],
                                      bufA.at[pl.ds(j * CH, CH)],
                                      gsA.at[j]).wait()
                pltpu.make_async_copy(dp_hbm.at[idxd.at[0]],
                                      bufB.at[pl.ds(j * CH, CH)],
                                      gsB.at[j]).wait()
            for j in range(NB):
                ch = sidx * NB + j
                off = base + ch * CH
                pltpu.sync_copy(bufA.at[pl.ds(j * CH, CH)],
                                acc.at[pl.ds(sbase + j * CH, CH)])
                pltpu.sync_copy(bufB.at[pl.ds(j * CH, CH)],
                                acc.at[identb.at[pl.ds(j * CH, CH)]], add=True)
                pltpu.sync_copy(acc.at[pl.ds(sbase + j * CH, CH)],
                                out_hbm.at[pl.ds(off, CH)])

                @pl.when(sidx < NSUP - 1)
                def _():
                    nch = ch + NB
                    pltpu.async_copy(sp_hbm.at[idxs.at[nch]],
                                     bufA.at[pl.ds(j * CH, CH)], gsA.at[j])
                    pltpu.async_copy(dp_hbm.at[idxd.at[nch]],
                                     bufB.at[pl.ds(j * CH, CH)], gsB.at[j])
            return carry

        lax.fori_loop(0, NSUP, super_body, 0)
        for t in range(NTAIL):
            ch = NSUP * NB + t
            off = base + ch * CH
            cpA = pltpu.async_copy(sp_hbm.at[idxs.at[ch]],
                                   bufA.at[pl.ds(0, CH)], gsA.at[0])
            cpB = pltpu.async_copy(dp_hbm.at[idxd.at[ch]],
                                   bufB.at[pl.ds(0, CH)], gsB.at[0])
            cpA.wait()
            cpB.wait()
            pltpu.sync_copy(bufA.at[pl.ds(0, CH)], acc.at[pl.ds(sbase, CH)])
            pltpu.sync_copy(bufB.at[pl.ds(0, CH)],
                            acc.at[identb.at[pl.ds(0, CH)]], add=True)
            pltpu.sync_copy(acc.at[pl.ds(sbase, CH)], out_hbm.at[pl.ds(off, CH)])

    return sc1


def _make_tc2_body(D, W):
    def body(g_ref, ef_ref, w1c_ref, w2_ref, w3_ref, e_ref, p_ref):
        x = g_ref[...] + jnp.dot(ef_ref[...], w1c_ref[...],
                                 preferred_element_type=jnp.float32)
        e = _leaky(x, 1e-4)
        e_ref[...] = e
        w3e = _leaky(jnp.dot(e, w3_ref[...], preferred_element_type=jnp.float32),
                     0.01)
        lg = _leaky(jnp.dot(e, w2_ref[...], preferred_element_type=jnp.float32),
                    1e-4)
        ex = jnp.exp(lg)
        p_ref[:, :D] = w3e * ex
        p_ref[:, D:D + 1] = ex
        p_ref[:, D + 1:] = jnp.zeros((ex.shape[0], W - D - 1), jnp.float32)
    return body


def _make_tc2(E, D, W, blk):
    grid = E // blk
    return pl.pallas_call(
        _make_tc2_body(D, W),
        grid=(grid,),
        in_specs=[
            pl.BlockSpec((blk, D), lambda i: (i, 0)),
            pl.BlockSpec((blk, D), lambda i: (i, 0)),
            pl.BlockSpec((D, D), lambda i: (0, 0)),
            pl.BlockSpec((D, 1), lambda i: (0, 0)),
            pl.BlockSpec((D, D), lambda i: (0, 0)),
        ],
        out_specs=[
            pl.BlockSpec((blk, D), lambda i: (i, 0)),
            pl.BlockSpec((blk, W), lambda i: (i, 0)),
        ],
        out_shape=[
            jax.ShapeDtypeStruct((E, D), jnp.float32),
            jax.ShapeDtypeStruct((E, W), jnp.float32),
        ],
    )


def _make_sc2(E, N, D, W, NC, NS):
    NW = NC * NS
    EPW = E // NW
    CH, NB = _CH, 2
    NCHUNK = EPW // CH
    NSUP = NCHUNK // NB
    NTAIL = NCHUNK - NSUP * NB
    NCH_N = N // CH
    assert N % CH == 0
    mesh = plsc.VectorSubcoreMesh(core_axis_name="c", subcore_axis_name="s")

    @functools.partial(
        pl.kernel,
        mesh=mesh,
        out_type=jax.ShapeDtypeStruct((NC, N, W), jnp.float32),
        compiler_params=pltpu.CompilerParams(use_tc_tiling_on_sc=False,
                                             needs_layout_passes=False),
        scratch_types=[
            pltpu.VMEM((EPW,), jnp.int32),
            pltpu.VMEM((NB * CH, W), jnp.float32),
            pltpu.VMEM_SHARED((N, W), jnp.float32),
            pltpu.SemaphoreType.DMA((NB,)),
        ],
    )
    def sc2(p_hbm, dst_hbm, out_hbm, idxd, bufP, mtab, lsem):
        c = lax.axis_index("c")
        s = lax.axis_index("s")
        wid = s * NC + c
        base = wid * EPW
        nstripe = (NCH_N - s + NS - 1) // NS

        pltpu.sync_copy(dst_hbm.at[wid], idxd)

        zf = jnp.zeros((16,), jnp.float32)

        def zrow(r, carry):
            for k in range(W // 16):
                bufP[r, pl.ds(k * 16, 16)] = zf
            return carry

        lax.fori_loop(0, CH, zrow, 0)

        def zstep(j, carry):
            pltpu.sync_copy(bufP.at[pl.ds(0, CH)],
                            mtab.at[pl.ds((s + j * NS) * CH, CH)])
            return carry

        lax.fori_loop(0, nstripe, zstep, 0)
        plsc.subcore_barrier()

        for j in range(NB):
            pltpu.async_copy(p_hbm.at[pl.ds(base + j * CH, CH)],
                             bufP.at[pl.ds(j * CH, CH)], lsem.at[j])

        def super_body(sidx, carry):
            for j in range(NB):
                ch = sidx * NB + j
                pltpu.make_async_copy(p_hbm.at[pl.ds(base, CH)],
                                      bufP.at[pl.ds(j * CH, CH)],
                                      lsem.at[j]).wait()
                pltpu.sync_copy(bufP.at[pl.ds(j * CH, CH)],
                                mtab.at[idxd.at[pl.ds(ch * CH, CH)]],
                                add=True)

                @pl.when(sidx < NSUP - 1)
                def _():
                    off = base + (ch + NB) * CH
                    pltpu.async_copy(p_hbm.at[pl.ds(off, CH)],
                                     bufP.at[pl.ds(j * CH, CH)], lsem.at[j])
            return carry

        lax.fori_loop(0, NSUP, super_body, 0)
        for t in range(NTAIL):
            ch = NSUP * NB + t
            off = base + ch * CH
            pltpu.sync_copy(p_hbm.at[pl.ds(off, CH)], bufP.at[pl.ds(0, CH)])
            pltpu.sync_copy(bufP.at[pl.ds(0, CH)],
                            mtab.at[idxd.at[pl.ds(ch * CH, CH)]], add=True)
        plsc.subcore_barrier()

        def wstep(j, carry):
            r0 = (s + j * NS) * CH
            pltpu.sync_copy(mtab.at[pl.ds(r0, CH)],
                            out_hbm.at[c, pl.ds(r0, CH)])
            return carry

        lax.fori_loop(0, nstripe, wstep, 0)

    return sc2


def _make_tc3_body(D):
    def body(m0_ref, m1_ref, nf_ref, wih_ref, whh_ref, bih_ref, bhh_ref, h_ref):
        msum = m0_ref[:, :D] + m1_ref[:, :D]
        den = m0_ref[:, D:D + 1] + m1_ref[:, D:D + 1]
        mdiv = jnp.where(den > 0, msum / jnp.where(den > 0, den, 1.0), 0.0)
        m = jnp.where(mdiv > 0, mdiv, jnp.exp(mdiv) - 1.0)
        nf = nf_ref[...]
        gi = jnp.dot(m, wih_ref[...], preferred_element_type=jnp.float32) + bih_ref[...]
        gh = jnp.dot(nf, whh_ref[...], preferred_element_type=jnp.float32) + bhh_ref[...]
        r = jax.nn.sigmoid(gi[:, :D] + gh[:, :D])
        z = jax.nn.sigmoid(gi[:, D:2 * D] + gh[:, D:2 * D])
        n = jnp.tanh(gi[:, 2 * D:] + r * gh[:, 2 * D:])
        h_ref[...] = jnp.maximum((1.0 - z) * n + z * nf, 0.0)
    return body


def _make_tc3(N, D, W, blk):
    grid = N // blk
    return pl.pallas_call(
        _make_tc3_body(D),
        grid=(grid,),
        in_specs=[
            pl.BlockSpec((blk, W), lambda i: (i, 0)),
            pl.BlockSpec((blk, W), lambda i: (i, 0)),
            pl.BlockSpec((blk, D), lambda i: (i, 0)),
            pl.BlockSpec((D, 3 * D), lambda i: (0, 0)),
            pl.BlockSpec((D, 3 * D), lambda i: (0, 0)),
            pl.BlockSpec((1, 3 * D), lambda i: (0, 0)),
            pl.BlockSpec((1, 3 * D), lambda i: (0, 0)),
        ],
        out_specs=pl.BlockSpec((blk, D), lambda i: (i, 0)),
        out_shape=jax.ShapeDtypeStruct((N, D), jnp.float32),
    )


def kernel(nfeats, efeats, edge_index, W1, W2, W3, W_ih, W_hh, b_ih, b_hh):
    N, D = nfeats.shape
    E = efeats.shape[0]
    W = D + 16
    info = plsc.get_sparse_core_info()
    NC, NS = info.num_cores, info.num_subcores
    NW = NC * NS
    NCHUNK = (E // NW) // _CH

    src = edge_index[0].reshape(NW, NCHUNK, _CH)
    dst = edge_index[1].reshape(NW, NCHUNK, _CH)

    sp, dp = _make_tc1(N, D, 2000)(nfeats, W1[:D], W1[D:2 * D])
    g = _make_sc1(E, D, NC, NS)(sp, dp, src, dst)
    e, p = _make_tc2(E, D, W, 3200)(g, efeats, W1[2 * D:], W2, W3)
    mtab = _make_sc2(E, N, D, W, NC, NS)(p, edge_index[1].reshape(NW, E // NW))
    h_new = _make_tc3(N, D, W, 2000)(
        mtab[0], mtab[1], nfeats, W_ih, W_hh,
        b_ih.reshape(1, 3 * D), b_hh.reshape(1, 3 * D))
    return (h_new, e)
